# baseline (speedup 1.0000x reference)
import jax
import jax.numpy as jnp
from jax import lax
from jax.experimental import pallas as pl
from jax.experimental.pallas import tpu as pltpu
from jax.experimental.pallas import tpu_sc as plsc

C0, C1 = 20000, 60000
D0, D1, D2 = 1024, 256, 64
D2P = 2 * D2
OUT_DIM = 1024

NC, NS = 2, 16
NW = NC * NS
TPW = 640
CH2 = 128
N2 = TPW // CH2

BMC = 128
CAP = 768
CPW = CAP // BMC
GCAP = NW * CAP


def _gather2_sc(tok, emb2):
    T = tok.shape[0]

    def body(tok_hbm, e2_hbm, e2out_hbm, tok_v, i2_v, b2a, b2b, sem_g, sem_w):
        wid = lax.axis_index("s") * NC + lax.axis_index("c")
        base = wid * TPW
        pltpu.sync_copy(tok_hbm.at[pl.ds(base, TPW)], tok_v)
        zero = jnp.zeros((16,), jnp.int32)
        for j in range(TPW // 16):
            t = tok_v[pl.ds(j * 16, 16)]
            i2 = jnp.where(t >= C1, t - C1, zero)
            i2_v[j // (CH2 // 16), pl.ds((j % (CH2 // 16)) * 16, 16)] = i2

        def sg(c):
            return pltpu.async_copy(
                e2_hbm.at[i2_v.at[c]], (b2a, b2b)[c % 2], sem_g)

        def sw(c):
            return pltpu.async_copy(
                (b2a, b2b)[c % 2],
                e2out_hbm.at[pl.ds(base + c * CH2, CH2)], sem_w)

        cg = {0: sg(0), 1: sg(1)}
        cw = {}
        for c in range(N2):
            cg[c].wait()
            cw[c] = sw(c)
            if c + 2 < N2:
                cw[c].wait()
                cg[c + 2] = sg(c + 2)
        cw[N2 - 2].wait()
        cw[N2 - 1].wait()

    run = pl.kernel(
        body,
        out_type=jax.ShapeDtypeStruct((T, 2 * D2), jnp.float32),
        mesh=plsc.VectorSubcoreMesh(core_axis_name="c", subcore_axis_name="s"),
        compiler_params=pltpu.CompilerParams(
            needs_layout_passes=False, use_tc_tiling_on_sc=False),
        scratch_types=[
            pltpu.VMEM((TPW,), jnp.int32),
            pltpu.VMEM((N2, CH2), jnp.int32),
            pltpu.VMEM((CH2, 2 * D2), jnp.float32),
            pltpu.VMEM((CH2, 2 * D2), jnp.float32),
            pltpu.SemaphoreType.DMA,
            pltpu.SemaphoreType.DMA,
        ],
    )
    return run(tok, emb2)


def _route01_sc(tok, emb0, emb1):
    def body(tok_hbm, e0_hbm, e1_hbm,
             g0_hbm, g1_hbm, pos0_hbm, pos1_hbm, cnt_hbm,
             tok_v, lp0_v, li0_v, lp1_v, li1_v,
             gb0_v, gb1_v, st_v, sem_g):
        wid = lax.axis_index("s") * NC + lax.axis_index("c")
        base = wid * TPW
        seg = wid * CAP
        pltpu.sync_copy(tok_hbm.at[pl.ds(base, TPW)], tok_v)

        iota = lax.iota(jnp.int32, 16)
        zero = jnp.zeros((16,), jnp.int32)
        for j in range((TPW + 16) // 16):
            li0_v[pl.ds(j * 16, 16)] = zero
            li1_v[pl.ds(j * 16, 16)] = zero
        n0v = zero
        n1v = zero
        for j in range(TPW // 16):
            t = tok_v[pl.ds(j * 16, 16)]
            m0 = t < C0
            m1 = jnp.logical_and(t >= C0, t < C1)
            pos = base + j * 16 + iota
            d0 = n0v + plsc.cumsum(m0.astype(jnp.int32)) - 1
            plsc.store_scatter(lp0_v, [d0], pos, mask=m0)
            plsc.store_scatter(li0_v, [d0], t, mask=m0)
            n0v = n0v + plsc.all_reduce_population_count(m0)
            d1 = n1v + plsc.cumsum(m1.astype(jnp.int32)) - 1
            plsc.store_scatter(lp1_v, [d1], pos, mask=m1)
            plsc.store_scatter(li1_v, [d1], t - C0, mask=m1)
            n1v = n1v + plsc.all_reduce_population_count(m1)

        cvec = jnp.where(iota == 0, n0v, jnp.where(iota == 1, n1v, zero))
        st_v[...] = cvec
        pltpu.sync_copy(st_v, cnt_hbm.at[pl.ds(wid * 16, 16)])

        def wpos(k):
            pltpu.sync_copy(lp0_v.at[pl.ds(k * 16, 16)],
                            pos0_hbm.at[pl.ds(seg + k * 16, 16)])
            pltpu.async_copy(
                e0_hbm.at[li0_v.at[pl.ds(k * 16, 16)]], gb0_v, sem_g).wait()
            pltpu.sync_copy(gb0_v, g0_hbm.at[pl.ds(seg + k * 16, 16)])
            return k + 1
        lax.while_loop(lambda k: jnp.any(n0v > k * 16), wpos, jnp.int32(0))

        def wpos1(k):
            pltpu.sync_copy(lp1_v.at[pl.ds(k * 16, 16)],
                            pos1_hbm.at[pl.ds(seg + k * 16, 16)])
            pltpu.async_copy(
                e1_hbm.at[li1_v.at[pl.ds(k * 16, 16)]], gb1_v, sem_g).wait()
            pltpu.sync_copy(gb1_v, g1_hbm.at[pl.ds(seg + k * 16, 16)])
            return k + 1
        lax.while_loop(lambda k: jnp.any(n1v > k * 16), wpos1, jnp.int32(0))

    run = pl.kernel(
        body,
        out_type=(
            jax.ShapeDtypeStruct((GCAP, D0), jnp.float32),
            jax.ShapeDtypeStruct((GCAP, D1), jnp.float32),
            jax.ShapeDtypeStruct((GCAP,), jnp.int32),
            jax.ShapeDtypeStruct((GCAP,), jnp.int32),
            jax.ShapeDtypeStruct((NW * 16,), jnp.int32),
        ),
        mesh=plsc.VectorSubcoreMesh(core_axis_name="c", subcore_axis_name="s"),
        compiler_params=pltpu.CompilerParams(needs_layout_passes=False),
        scratch_types=[
            pltpu.VMEM((TPW,), jnp.int32),
            pltpu.VMEM((TPW + 16,), jnp.int32),
            pltpu.VMEM((TPW + 16,), jnp.int32),
            pltpu.VMEM((TPW + 16,), jnp.int32),
            pltpu.VMEM((TPW + 16,), jnp.int32),
            pltpu.VMEM((16, D0), jnp.float32),
            pltpu.VMEM((16, D1), jnp.float32),
            pltpu.VMEM((16,), jnp.int32),
            pltpu.SemaphoreType.DMA,
        ],
    )
    return run(tok, emb0, emb1)


def _main_matmul_tc(e2, w2b):
    T = e2.shape[0]
    bm = 1024

    def body(e2_ref, w_ref, o_ref):
        o_ref[...] = lax.dot_general(
            e2_ref[...].astype(jnp.bfloat16), w_ref[...],
            (((1,), (1,)), ((), ())), preferred_element_type=jnp.float32)

    return pl.pallas_call(
        body,
        grid=(T // bm,),
        in_specs=[
            pl.BlockSpec((bm, 2 * D2), lambda i: (i, 0)),
            pl.BlockSpec((OUT_DIM, 2 * D2), lambda i: (0, 0)),
        ],
        out_specs=pl.BlockSpec((bm, OUT_DIM), lambda i: (i, 0)),
        out_shape=jax.ShapeDtypeStruct((T, OUT_DIM), jnp.float32),
    )(e2, w2b)


def _correct_tc(cnt, g0, g1, pos0, pos1, W0, W1, out2d):
    T = out2d.shape[0]

    def body(cnt_ref, g0_ref, g1_ref, p0_ref, p1_ref, w0_ref, w1_ref,
             oin_ref, o_ref, acc, pos_s, sem_p, sem_w):
        w = pl.program_id(0)
        j = pl.program_id(1)
        del oin_ref

        def band(cnt_w, g_ref, w_ref, pos_hbm):
            nrem = cnt_w - j * BMC

            @pl.when(nrem > 0)
            def _():
                row = w * CAP + j * BMC
                pcp = pltpu.make_async_copy(
                    pos_hbm.at[pl.ds(row, BMC)], pos_s, sem_p)
                pcp.start()
                acc[...] = lax.dot_general(
                    g_ref[...].astype(jnp.bfloat16), w_ref[...],
                    (((1,), (1,)), ((), ())),
                    preferred_element_type=jnp.float32)
                pcp.wait()
                nact = jnp.minimum(nrem, BMC)

                def issue(i, _):
                    p = pos_s[i]
                    pltpu.make_async_copy(
                        acc.at[pl.ds(i, 1)], o_ref.at[pl.ds(p, 1)],
                        sem_w).start()
                    return 0
                lax.fori_loop(0, nact, issue, 0)

                def drain(i, _):
                    pltpu.make_async_copy(
                        acc.at[pl.ds(0, 1)], o_ref.at[pl.ds(0, 1)],
                        sem_w).wait()
                    return 0
                lax.fori_loop(0, nact, drain, 0)

        band(cnt_ref[w, 0], g0_ref, w0_ref, p0_ref)
        band(cnt_ref[w, 1], g1_ref, w1_ref, p1_ref)

    return pl.pallas_call(
        body,
        grid=(NW, CPW),
        in_specs=[
            pl.BlockSpec(memory_space=pltpu.SMEM),
            pl.BlockSpec((BMC, D0), lambda w, j: (w * CPW + j, 0)),
            pl.BlockSpec((BMC, D1), lambda w, j: (w * CPW + j, 0)),
            pl.BlockSpec(memory_space=pltpu.HBM),
            pl.BlockSpec(memory_space=pltpu.HBM),
            pl.BlockSpec((OUT_DIM, D0), lambda w, j: (0, 0)),
            pl.BlockSpec((OUT_DIM, D1), lambda w, j: (0, 0)),
            pl.BlockSpec(memory_space=pltpu.HBM),
        ],
        out_specs=pl.BlockSpec(memory_space=pltpu.HBM),
        out_shape=jax.ShapeDtypeStruct((T, OUT_DIM), jnp.float32),
        input_output_aliases={7: 0},
        scratch_shapes=[
            pltpu.VMEM((BMC, OUT_DIM), jnp.float32),
            pltpu.SMEM((BMC,), jnp.int32),
            pltpu.SemaphoreType.DMA,
            pltpu.SemaphoreType.DMA,
        ],
    )(cnt, g0, g1, pos0, pos1,
      W0.astype(jnp.bfloat16), W1.astype(jnp.bfloat16), out2d)


def kernel(input, emb0, emb1, emb2, W0, W1, W2):
    B, L = input.shape
    T = B * L
    tok = jnp.transpose(input).reshape(T)
    g0, g1, pos0, pos1, cnt = _route01_sc(tok, emb0, emb1)
    e2 = _gather2_sc(tok, jnp.pad(emb2, ((0, 0), (0, D2))))
    w2padb = jnp.pad(W2, ((0, 0), (0, D2))).astype(jnp.bfloat16)
    out2d = _main_matmul_tc(e2, w2padb)
    out2d = _correct_tc(cnt.reshape(NW, 16), g0, g1, pos0, pos1,
                        W0, W1, out2d)
    out3 = out2d.reshape(L, B, OUT_DIM)
    return jnp.transpose(out3, (1, 0, 2))


# speedup vs baseline: 1.0766x; 1.0766x over previous
import jax
import jax.numpy as jnp
from jax import lax
from jax.experimental import pallas as pl
from jax.experimental.pallas import tpu as pltpu
from jax.experimental.pallas import tpu_sc as plsc

C0, C1 = 20000, 60000
D0, D1, D2 = 1024, 256, 64
D2P = 2 * D2
OUT_DIM = 1024

NC, NS = 2, 16
NW = NC * NS
TPW = 640
CH2 = 128
N2 = TPW // CH2

BMC = 256
CAP = 768
CPW = CAP // BMC
GCAP = NW * CAP


def _gather2_sc(tok, emb2):
    T = tok.shape[0]

    def body(tok_hbm, e2_hbm, e2out_hbm, tok_v, i2_v, b2a, b2b, sem_g, sem_w):
        wid = lax.axis_index("s") * NC + lax.axis_index("c")
        base = wid * TPW
        pltpu.sync_copy(tok_hbm.at[pl.ds(base, TPW)], tok_v)
        zero = jnp.zeros((16,), jnp.int32)
        for j in range(TPW // 16):
            t = tok_v[pl.ds(j * 16, 16)]
            i2 = jnp.where(t >= C1, t - C1, zero)
            i2_v[j // (CH2 // 16), pl.ds((j % (CH2 // 16)) * 16, 16)] = i2

        def sg(c):
            return pltpu.async_copy(
                e2_hbm.at[i2_v.at[c]], (b2a, b2b)[c % 2], sem_g)

        def sw(c):
            return pltpu.async_copy(
                (b2a, b2b)[c % 2],
                e2out_hbm.at[pl.ds(base + c * CH2, CH2)], sem_w)

        cg = {0: sg(0), 1: sg(1)}
        cw = {}
        for c in range(N2):
            cg[c].wait()
            cw[c] = sw(c)
            if c + 2 < N2:
                cw[c].wait()
                cg[c + 2] = sg(c + 2)
        cw[N2 - 2].wait()
        cw[N2 - 1].wait()

    run = pl.kernel(
        body,
        out_type=jax.ShapeDtypeStruct((T, 2 * D2), jnp.float32),
        mesh=plsc.VectorSubcoreMesh(core_axis_name="c", subcore_axis_name="s"),
        compiler_params=pltpu.CompilerParams(
            needs_layout_passes=False, use_tc_tiling_on_sc=False),
        scratch_types=[
            pltpu.VMEM((TPW,), jnp.int32),
            pltpu.VMEM((N2, CH2), jnp.int32),
            pltpu.VMEM((CH2, 2 * D2), jnp.float32),
            pltpu.VMEM((CH2, 2 * D2), jnp.float32),
            pltpu.SemaphoreType.DMA,
            pltpu.SemaphoreType.DMA,
        ],
    )
    return run(tok, emb2)


def _route01_sc(tok, emb0, emb1):
    def body(tok_hbm, e0_hbm, e1_hbm,
             g0_hbm, g1_hbm, pos0_hbm, pos1_hbm, cnt_hbm,
             tok_v, lp0_v, li0_v, lp1_v, li1_v,
             gb0_v, gb1_v, st_v, sem_g):
        wid = lax.axis_index("s") * NC + lax.axis_index("c")
        base = wid * TPW
        seg = wid * CAP
        pltpu.sync_copy(tok_hbm.at[pl.ds(base, TPW)], tok_v)

        iota = lax.iota(jnp.int32, 16)
        zero = jnp.zeros((16,), jnp.int32)
        for j in range((TPW + 16) // 16):
            li0_v[pl.ds(j * 16, 16)] = zero
            li1_v[pl.ds(j * 16, 16)] = zero
        n0v = zero
        n1v = zero
        for j in range(TPW // 16):
            t = tok_v[pl.ds(j * 16, 16)]
            m0 = t < C0
            m1 = jnp.logical_and(t >= C0, t < C1)
            pos = base + j * 16 + iota
            d0 = n0v + plsc.cumsum(m0.astype(jnp.int32)) - 1
            plsc.store_scatter(lp0_v, [d0], pos, mask=m0)
            plsc.store_scatter(li0_v, [d0], t, mask=m0)
            n0v = n0v + plsc.all_reduce_population_count(m0)
            d1 = n1v + plsc.cumsum(m1.astype(jnp.int32)) - 1
            plsc.store_scatter(lp1_v, [d1], pos, mask=m1)
            plsc.store_scatter(li1_v, [d1], t - C0, mask=m1)
            n1v = n1v + plsc.all_reduce_population_count(m1)

        cvec = jnp.where(iota == 0, n0v, jnp.where(iota == 1, n1v, zero))
        st_v[...] = cvec
        pltpu.sync_copy(st_v, cnt_hbm.at[pl.ds(wid * 16, 16)])

        def wpos(k):
            pltpu.sync_copy(lp0_v.at[pl.ds(k * 16, 16)],
                            pos0_hbm.at[pl.ds(seg + k * 16, 16)])
            pltpu.async_copy(
                e0_hbm.at[li0_v.at[pl.ds(k * 16, 16)]], gb0_v, sem_g).wait()
            pltpu.sync_copy(gb0_v, g0_hbm.at[pl.ds(seg + k * 16, 16)])
            return k + 1
        lax.while_loop(lambda k: jnp.any(n0v > k * 16), wpos, jnp.int32(0))

        def wpos1(k):
            pltpu.sync_copy(lp1_v.at[pl.ds(k * 16, 16)],
                            pos1_hbm.at[pl.ds(seg + k * 16, 16)])
            pltpu.async_copy(
                e1_hbm.at[li1_v.at[pl.ds(k * 16, 16)]], gb1_v, sem_g).wait()
            pltpu.sync_copy(gb1_v, g1_hbm.at[pl.ds(seg + k * 16, 16)])
            return k + 1
        lax.while_loop(lambda k: jnp.any(n1v > k * 16), wpos1, jnp.int32(0))

    run = pl.kernel(
        body,
        out_type=(
            jax.ShapeDtypeStruct((GCAP, D0), jnp.float32),
            jax.ShapeDtypeStruct((GCAP, D1), jnp.float32),
            jax.ShapeDtypeStruct((GCAP,), jnp.int32),
            jax.ShapeDtypeStruct((GCAP,), jnp.int32),
            jax.ShapeDtypeStruct((NW * 16,), jnp.int32),
        ),
        mesh=plsc.VectorSubcoreMesh(core_axis_name="c", subcore_axis_name="s"),
        compiler_params=pltpu.CompilerParams(needs_layout_passes=False),
        scratch_types=[
            pltpu.VMEM((TPW,), jnp.int32),
            pltpu.VMEM((TPW + 16,), jnp.int32),
            pltpu.VMEM((TPW + 16,), jnp.int32),
            pltpu.VMEM((TPW + 16,), jnp.int32),
            pltpu.VMEM((TPW + 16,), jnp.int32),
            pltpu.VMEM((16, D0), jnp.float32),
            pltpu.VMEM((16, D1), jnp.float32),
            pltpu.VMEM((16,), jnp.int32),
            pltpu.SemaphoreType.DMA,
        ],
    )
    return run(tok, emb0, emb1)


def _main_matmul_tc(e2, w2b):
    T = e2.shape[0]
    bm = 1024

    def body(e2_ref, w_ref, o_ref):
        o_ref[...] = lax.dot_general(
            e2_ref[...].astype(jnp.bfloat16), w_ref[...],
            (((1,), (1,)), ((), ())), preferred_element_type=jnp.float32)

    return pl.pallas_call(
        body,
        grid=(T // bm,),
        in_specs=[
            pl.BlockSpec((bm, 2 * D2), lambda i: (i, 0)),
            pl.BlockSpec((OUT_DIM, 2 * D2), lambda i: (0, 0)),
        ],
        out_specs=pl.BlockSpec((bm, OUT_DIM), lambda i: (i, 0)),
        out_shape=jax.ShapeDtypeStruct((T, OUT_DIM), jnp.float32),
    )(e2, w2b)


def _correct_tc(cnt, g0, g1, pos0, pos1, W0, W1, out2d):
    T = out2d.shape[0]

    def body(cnt_ref, g0_ref, g1_ref, p0_ref, p1_ref, w0_ref, w1_ref,
             oin_ref, o_ref, acc, pos_s, sem_p, sem_w):
        w = pl.program_id(0)
        j = pl.program_id(1)
        del oin_ref

        def band(cnt_w, g_ref, w_ref, pos_hbm):
            nrem = cnt_w - j * BMC

            @pl.when(nrem > 0)
            def _():
                row = w * CAP + j * BMC
                pcp = pltpu.make_async_copy(
                    pos_hbm.at[pl.ds(row, BMC)], pos_s, sem_p)
                pcp.start()
                acc[...] = lax.dot_general(
                    g_ref[...].astype(jnp.bfloat16), w_ref[...],
                    (((1,), (1,)), ((), ())),
                    preferred_element_type=jnp.float32)
                pcp.wait()
                nact = jnp.minimum(nrem, BMC)

                def issue(i, _):
                    p = pos_s[i]
                    pltpu.make_async_copy(
                        acc.at[pl.ds(i, 1)], o_ref.at[pl.ds(p, 1)],
                        sem_w).start()
                    return 0
                lax.fori_loop(0, nact, issue, 0)

                def drain(i, _):
                    pltpu.make_async_copy(
                        acc.at[pl.ds(0, 1)], o_ref.at[pl.ds(0, 1)],
                        sem_w).wait()
                    return 0
                lax.fori_loop(0, nact, drain, 0)

        band(cnt_ref[w, 0], g0_ref, w0_ref, p0_ref)
        band(cnt_ref[w, 1], g1_ref, w1_ref, p1_ref)

    return pl.pallas_call(
        body,
        grid=(NW, CPW),
        in_specs=[
            pl.BlockSpec(memory_space=pltpu.SMEM),
            pl.BlockSpec((BMC, D0), lambda w, j: (w * CPW + j, 0)),
            pl.BlockSpec((BMC, D1), lambda w, j: (w * CPW + j, 0)),
            pl.BlockSpec(memory_space=pltpu.HBM),
            pl.BlockSpec(memory_space=pltpu.HBM),
            pl.BlockSpec((OUT_DIM, D0), lambda w, j: (0, 0)),
            pl.BlockSpec((OUT_DIM, D1), lambda w, j: (0, 0)),
            pl.BlockSpec(memory_space=pltpu.HBM),
        ],
        out_specs=pl.BlockSpec(memory_space=pltpu.HBM),
        out_shape=jax.ShapeDtypeStruct((T, OUT_DIM), jnp.float32),
        input_output_aliases={7: 0},
        scratch_shapes=[
            pltpu.VMEM((BMC, OUT_DIM), jnp.float32),
            pltpu.SMEM((BMC,), jnp.int32),
            pltpu.SemaphoreType.DMA,
            pltpu.SemaphoreType.DMA,
        ],
    )(cnt, g0, g1, pos0, pos1,
      W0.astype(jnp.bfloat16), W1.astype(jnp.bfloat16), out2d)


def kernel(input, emb0, emb1, emb2, W0, W1, W2):
    B, L = input.shape
    T = B * L
    tok = jnp.transpose(input).reshape(T)
    g0, g1, pos0, pos1, cnt = _route01_sc(tok, emb0, emb1)
    e2 = _gather2_sc(tok, jnp.pad(emb2, ((0, 0), (0, D2))))
    w2padb = jnp.pad(W2, ((0, 0), (0, D2))).astype(jnp.bfloat16)
    out2d = _main_matmul_tc(e2, w2padb)
    out2d = _correct_tc(cnt.reshape(NW, 16), g0, g1, pos0, pos1,
                        W0, W1, out2d)
    out3 = out2d.reshape(L, B, OUT_DIM)
    return jnp.transpose(out3, (1, 0, 2))
